# baseline (speedup 1.0000x reference)
import jax
import jax.numpy as jnp
from jax.experimental import pallas as pl
from jax.experimental.pallas import tpu as pltpu

_EPS = 1e-5
_K = 8


def _ln(x, gamma, beta):
    mean = jnp.mean(x, axis=-1, keepdims=True)
    var = jnp.mean((x - mean) ** 2, axis=-1, keepdims=True)
    return (x - mean) * jax.lax.rsqrt(var + _EPS) * gamma + beta


def _scale_body(tv_ref, tt_ref, vt_ref, tg_ref, tb_ref, vg_ref, vb_ref,
                scale_t_ref):
    B = tv_ref.shape[0]
    V = tt_ref.shape[0]
    idx = tv_ref[:]
    iota = jax.lax.broadcasted_iota(jnp.int32, (B, V), 1)
    oh_t = (iota == idx[:, 0:1]).astype(jnp.float32)
    oh_v = (iota == idx[:, 1:2]).astype(jnp.float32)
    temb = jnp.dot(oh_t, tt_ref[:], preferred_element_type=jnp.float32,
                   precision=jax.lax.Precision.HIGHEST)
    vemb = jnp.dot(oh_v, vt_ref[:], preferred_element_type=jnp.float32,
                   precision=jax.lax.Precision.HIGHEST)
    tln = _ln(temb, tg_ref[:], tb_ref[:])
    vln = _ln(vemb, vg_ref[:], vb_ref[:])
    scale_t_ref[:] = tln * vln


def _mul_body(scale_t_ref, ft_ref, out_ref, in_buf, out_buf, in_sems,
              out_sems):
    N = ft_ref.shape[0]

    def in_copy(i, slot):
        return pltpu.make_async_copy(
            ft_ref.at[pl.ds(i, 1)], in_buf.at[pl.ds(slot, 1)],
            in_sems.at[slot])

    def out_copy(i, slot):
        return pltpu.make_async_copy(
            out_buf.at[pl.ds(slot, 1)], out_ref.at[pl.ds(i, 1)],
            out_sems.at[slot])

    for k in range(_K - 1):
        in_copy(k, k).start()

    def step(i, carry):
        slot = jax.lax.rem(i, _K)
        nxt = i + _K - 1

        @pl.when(nxt < N)
        def _():
            in_copy(nxt, jax.lax.rem(nxt, _K)).start()

        in_copy(i, slot).wait()

        @pl.when(i >= _K)
        def _():
            out_copy(i - _K, slot).wait()

        col = scale_t_ref[pl.ds(i, 1)]
        out_buf[pl.ds(slot, 1)] = in_buf[pl.ds(slot, 1)] * col
        out_copy(i, slot).start()
        return carry

    jax.lax.fori_loop(0, N, step, 0)

    for i in range(max(0, N - _K), N):
        out_copy(i, i % _K).wait()


def kernel(ft, taskvar, task_table, var_table, task_gamma, task_beta,
           var_gamma, var_beta):
    B, C, H, W = ft.shape
    HW = H * W

    scale = pl.pallas_call(
        _scale_body,
        out_shape=jax.ShapeDtypeStruct((B, C), jnp.float32),
    )(taskvar, task_table, var_table,
      task_gamma.reshape(1, C), task_beta.reshape(1, C),
      var_gamma.reshape(1, C), var_beta.reshape(1, C))
    scale_cols = scale.reshape(B, C, 1)

    ft3 = ft.reshape(B, C, HW)
    out3 = pl.pallas_call(
        _mul_body,
        in_specs=[
            pl.BlockSpec(memory_space=pltpu.MemorySpace.VMEM),
            pl.BlockSpec(memory_space=pl.ANY),
        ],
        out_specs=pl.BlockSpec(memory_space=pl.ANY),
        out_shape=jax.ShapeDtypeStruct((B, C, HW), ft.dtype),
        scratch_shapes=[
            pltpu.VMEM((_K, C, HW), jnp.float32),
            pltpu.VMEM((_K, C, HW), jnp.float32),
            pltpu.SemaphoreType.DMA((_K,)),
            pltpu.SemaphoreType.DMA((_K,)),
        ],
    )(scale_cols, ft3)
    return out3.reshape(B, C, H, W)
